# baseline (speedup 1.0000x reference)
import jax
import jax.numpy as jnp
from jax.experimental import pallas as pl
from jax.experimental.pallas import tpu as pltpu

N = 10000
NFEAT = 128
NHID = 128
NCLASS = 64

BI = 400
GRID = N // BI


def _s1_kernel(x_ref, w1_ref, s1_ref):
    s1_ref[...] = jnp.dot(x_ref[...], w1_ref[...],
                          preferred_element_type=jnp.float32)


def _layer1_kernel(adj_ref, s1_ref, b1_ref, w2_ref, s2_ref):
    part = jnp.dot(adj_ref[...], s1_ref[...],
                   preferred_element_type=jnp.float32)
    h = jnp.maximum(part + b1_ref[...], 0.0)
    s2_ref[...] = jnp.dot(h, w2_ref[...], preferred_element_type=jnp.float32)


def _layer2_kernel(adj_ref, s2_ref, b2_ref, out_ref):
    o = jnp.dot(adj_ref[...], s2_ref[...],
                preferred_element_type=jnp.float32) + b2_ref[...]
    m = jnp.max(o, axis=1, keepdims=True)
    shifted = o - m
    lse = jnp.log(jnp.sum(jnp.exp(shifted), axis=1, keepdims=True))
    out_ref[...] = shifted - lse


@jax.jit
def kernel(x, adj, W1, b1, W2, b2):
    b1r = b1.reshape(1, NHID)
    b2r = b2.reshape(1, NCLASS)

    s1 = pl.pallas_call(
        _s1_kernel,
        in_specs=[
            pl.BlockSpec((N, NFEAT), lambda: (0, 0)),
            pl.BlockSpec((NFEAT, NHID), lambda: (0, 0)),
        ],
        out_specs=pl.BlockSpec((N, NHID), lambda: (0, 0)),
        out_shape=jax.ShapeDtypeStruct((N, NHID), jnp.float32),
    )(x, W1)

    s2 = pl.pallas_call(
        _layer1_kernel,
        grid=(GRID,),
        in_specs=[
            pl.BlockSpec((BI, N), lambda i: (i, 0)),
            pl.BlockSpec((N, NHID), lambda i: (0, 0)),
            pl.BlockSpec((1, NHID), lambda i: (0, 0)),
            pl.BlockSpec((NHID, NCLASS), lambda i: (0, 0)),
        ],
        out_specs=pl.BlockSpec((BI, NCLASS), lambda i: (i, 0)),
        out_shape=jax.ShapeDtypeStruct((N, NCLASS), jnp.float32),
        compiler_params=pltpu.CompilerParams(
            dimension_semantics=("arbitrary",),
        ),
    )(adj, s1, b1r, W2)

    out = pl.pallas_call(
        _layer2_kernel,
        grid=(GRID,),
        in_specs=[
            pl.BlockSpec((BI, N), lambda i: (i, 0)),
            pl.BlockSpec((N, NCLASS), lambda i: (0, 0)),
            pl.BlockSpec((1, NCLASS), lambda i: (0, 0)),
        ],
        out_specs=pl.BlockSpec((BI, NCLASS), lambda i: (i, 0)),
        out_shape=jax.ShapeDtypeStruct((N, NCLASS), jnp.float32),
        compiler_params=pltpu.CompilerParams(
            dimension_semantics=("arbitrary",),
        ),
    )(adj, s2, b2r)

    return out


# speedup vs baseline: 1.0533x; 1.0533x over previous
import jax
import jax.numpy as jnp
from jax.experimental import pallas as pl
from jax.experimental.pallas import tpu as pltpu

N = 10000
NFEAT = 128
NHID = 128
NCLASS = 64

BI = 400
GRID = N // BI


def _gcn_kernel(x_ref, adj_ref, w1_ref, b1_ref, w2_ref, b2_ref,
                out_ref, s1_ref, s2_ref):
    g = pl.program_id(0)

    @pl.when(g == 0)
    def _phase0():
        s1_ref[...] = jnp.dot(x_ref[...], w1_ref[...],
                              preferred_element_type=jnp.float32)

    @pl.when((g >= 1) & (g <= GRID))
    def _phase1():
        i = g - 1
        part = jnp.dot(adj_ref[...], s1_ref[...],
                       preferred_element_type=jnp.float32)
        h = jnp.maximum(part + b1_ref[...], 0.0)
        s2_ref[pl.ds(i * BI, BI), :] = jnp.dot(
            h, w2_ref[...], preferred_element_type=jnp.float32)

    @pl.when(g > GRID)
    def _phase2():
        o = jnp.dot(adj_ref[...], s2_ref[...],
                    preferred_element_type=jnp.float32) + b2_ref[...]
        m = jnp.max(o, axis=1, keepdims=True)
        shifted = o - m
        lse = jnp.log(jnp.sum(jnp.exp(shifted), axis=1, keepdims=True))
        out_ref[...] = shifted - lse


def _adj_index(g):
    i1 = jnp.maximum(g - 1, 0)
    i2 = g - (GRID + 1)
    return (jnp.where(g > GRID, i2, i1), 0)


def _out_index(g):
    return (jnp.maximum(g - (GRID + 1), 0), 0)


@jax.jit
def kernel(x, adj, W1, b1, W2, b2):
    b1r = b1.reshape(1, NHID)
    b2r = b2.reshape(1, NCLASS)

    out = pl.pallas_call(
        _gcn_kernel,
        grid=(1 + 2 * GRID,),
        in_specs=[
            pl.BlockSpec((N, NFEAT), lambda g: (0, 0)),
            pl.BlockSpec((BI, N), _adj_index),
            pl.BlockSpec((NFEAT, NHID), lambda g: (0, 0)),
            pl.BlockSpec((1, NHID), lambda g: (0, 0)),
            pl.BlockSpec((NHID, NCLASS), lambda g: (0, 0)),
            pl.BlockSpec((1, NCLASS), lambda g: (0, 0)),
        ],
        out_specs=pl.BlockSpec((BI, NCLASS), _out_index),
        out_shape=jax.ShapeDtypeStruct((N, NCLASS), jnp.float32),
        scratch_shapes=[
            pltpu.VMEM((N, NHID), jnp.float32),
            pltpu.VMEM((N, NCLASS), jnp.float32),
        ],
        compiler_params=pltpu.CompilerParams(
            dimension_semantics=("arbitrary",),
        ),
    )(x, adj, W1, b1r, W2, b2r)

    return out
